# baseline (speedup 1.0000x reference)
import functools

import jax
import jax.numpy as jnp
from jax import lax
from jax.experimental import pallas as pl
from jax.experimental.pallas import tpu as pltpu
from jax.experimental.pallas import tpu_sc as plsc

_BN_EPS = 1e-5


def _pick_bn(n, cap=2048):
    for bn in (2048, 2000, 1600, 1280, 1024, 1000, 800, 640, 512, 500,
               400, 320, 256, 250, 200, 160, 128, 100, 80, 64, 50, 40,
               32, 25, 16, 8):
        if bn <= cap and n % bn == 0 and bn % 8 == 0:
            return bn
    return n


def _row_spec(bn, d):
    return pl.BlockSpec((bn, d), lambda i: (i, 0))


def _full_spec(shape):
    return pl.BlockSpec(shape, lambda i: tuple(0 for _ in shape))


def _linear1(x, msg, w, b, eps, want_stats):
    n, din = x.shape
    h = w.shape[1]
    bn = _pick_bn(n)
    grid = n // bn

    has_msg = msg is not None
    has_b = b is not None
    has_eps = eps is not None

    def body(*refs):
        it = iter(refs)
        x_ref = next(it)
        msg_ref = next(it) if has_msg else None
        w_ref = next(it)
        b_ref = next(it) if has_b else None
        eps_ref = next(it) if has_eps else None
        o_ref = next(it)
        st_ref = next(it) if want_stats else None
        acc_ref = next(it) if want_stats else None

        g = x_ref[...]
        if has_eps:
            g = (1.0 + eps_ref[0]) * g
        if has_msg:
            g = g + msg_ref[...]
        hh = jnp.dot(g, w_ref[...], preferred_element_type=jnp.float32)
        if has_b:
            hh = hh + b_ref[...]
        o_ref[...] = hh
        if want_stats:
            i = pl.program_id(0)

            @pl.when(i == 0)
            def _():
                acc_ref[...] = jnp.zeros_like(acc_ref)

            acc_ref[0:1, :] += jnp.sum(hh, axis=0, keepdims=True)
            acc_ref[1:2, :] += jnp.sum(hh * hh, axis=0, keepdims=True)

            @pl.when(i == grid - 1)
            def _():
                st_ref[...] = acc_ref[...]

    in_specs = [_row_spec(bn, din)]
    args = [x]
    if has_msg:
        in_specs.append(_row_spec(bn, din))
        args.append(msg)
    in_specs.append(_full_spec((din, h)))
    args.append(w)
    if has_b:
        in_specs.append(_full_spec((1, h)))
        args.append(b.reshape(1, h))
    if has_eps:
        in_specs.append(pl.BlockSpec(memory_space=pltpu.SMEM))
        args.append(eps.reshape(1))

    out_shape = [jax.ShapeDtypeStruct((n, h), jnp.float32)]
    out_specs = [_row_spec(bn, h)]
    scratch = []
    if want_stats:
        out_shape.append(jax.ShapeDtypeStruct((2, h), jnp.float32))
        out_specs.append(_full_spec((2, h)))
        scratch.append(pltpu.VMEM((2, h), jnp.float32))

    res = pl.pallas_call(
        body,
        grid=(grid,),
        in_specs=in_specs,
        out_specs=out_specs if len(out_specs) > 1 else out_specs,
        out_shape=out_shape,
        scratch_shapes=scratch,
    )(*args)
    if want_stats:
        return res[0], res[1]
    return res[0], None


def _bn_relu_linear(h1, stats, nrows, g, be, w, b, res=None, want_stats=False):
    n, h = h1.shape
    h2 = w.shape[1]
    bn = _pick_bn(n)
    grid = n // bn
    has_res = res is not None
    inv_n = 1.0 / float(nrows)

    def body(*refs):
        it = iter(refs)
        h_ref = next(it)
        st_ref = next(it)
        g_ref = next(it)
        be_ref = next(it)
        w_ref = next(it)
        b_ref = next(it)
        res_ref = next(it) if has_res else None
        o_ref = next(it)
        sto_ref = next(it) if want_stats else None
        acc_ref = next(it) if want_stats else None

        mu = st_ref[0:1, :] * inv_n
        var = st_ref[1:2, :] * inv_n - mu * mu
        scale = g_ref[...] * jax.lax.rsqrt(var + _BN_EPS)
        shift = be_ref[...] - mu * scale
        a = jnp.maximum(h_ref[...] * scale + shift, 0.0)
        out = jnp.dot(a, w_ref[...], preferred_element_type=jnp.float32)
        out = out + b_ref[...]
        if has_res:
            out = out + res_ref[...]
        o_ref[...] = out
        if want_stats:
            i = pl.program_id(0)

            @pl.when(i == 0)
            def _():
                acc_ref[...] = jnp.zeros_like(acc_ref)

            acc_ref[0:1, :] += jnp.sum(out, axis=0, keepdims=True)
            acc_ref[1:2, :] += jnp.sum(out * out, axis=0, keepdims=True)

            @pl.when(i == grid - 1)
            def _():
                sto_ref[...] = acc_ref[...]

    in_specs = [
        _row_spec(bn, h),
        _full_spec((2, h)),
        _full_spec((1, h)),
        _full_spec((1, h)),
        _full_spec((h, h2)),
        _full_spec((1, h2)),
    ]
    args = [h1, stats, g.reshape(1, h), be.reshape(1, h), w, b.reshape(1, h2)]
    if has_res:
        in_specs.append(_row_spec(bn, h2))
        args.append(res)

    out_shape = [jax.ShapeDtypeStruct((n, h2), jnp.float32)]
    out_specs = [_row_spec(bn, h2)]
    scratch = []
    if want_stats:
        out_shape.append(jax.ShapeDtypeStruct((2, h2), jnp.float32))
        out_specs.append(_full_spec((2, h2)))
        scratch.append(pltpu.VMEM((2, h2), jnp.float32))

    res_out = pl.pallas_call(
        body,
        grid=(grid,),
        in_specs=in_specs,
        out_specs=out_specs,
        out_shape=out_shape,
        scratch_shapes=scratch,
    )(*args)
    if want_stats:
        return res_out[0], res_out[1]
    return res_out[0], None


def _add_stats(a, b):
    n, h = a.shape
    bn = _pick_bn(n)
    grid = n // bn

    def body(a_ref, b_ref, o_ref, st_ref, acc_ref):
        out = a_ref[...] + b_ref[...]
        o_ref[...] = out
        i = pl.program_id(0)

        @pl.when(i == 0)
        def _():
            acc_ref[...] = jnp.zeros_like(acc_ref)

        acc_ref[0:1, :] += jnp.sum(out, axis=0, keepdims=True)
        acc_ref[1:2, :] += jnp.sum(out * out, axis=0, keepdims=True)

        @pl.when(i == grid - 1)
        def _():
            st_ref[...] = acc_ref[...]

    out, st = pl.pallas_call(
        body,
        grid=(grid,),
        in_specs=[_row_spec(bn, h), _row_spec(bn, h)],
        out_specs=[_row_spec(bn, h), _full_spec((2, h))],
        out_shape=[jax.ShapeDtypeStruct((n, h), jnp.float32),
                   jax.ShapeDtypeStruct((2, h), jnp.float32)],
        scratch_shapes=[pltpu.VMEM((2, h), jnp.float32)],
    )(a, b)
    return out, st


def _bn_relu_final(h2, stats, nrows, g, be, w3, b3):
    n, h = h2.shape
    bn = _pick_bn(n)
    grid = n // bn
    inv_n = 1.0 / float(nrows)

    def body(h_ref, st_ref, g_ref, be_ref, w_ref, b_ref, o_ref):
        mu = st_ref[0:1, :] * inv_n
        var = st_ref[1:2, :] * inv_n - mu * mu
        scale = g_ref[...] * jax.lax.rsqrt(var + _BN_EPS)
        shift = be_ref[...] - mu * scale
        a = jnp.maximum(h_ref[...] * scale + shift, 0.0)
        o_ref[...] = jnp.sum(a * w_ref[...], axis=1, keepdims=True) + b_ref[0]

    out = pl.pallas_call(
        body,
        grid=(grid,),
        in_specs=[
            _row_spec(bn, h),
            _full_spec((2, h)),
            _full_spec((1, h)),
            _full_spec((1, h)),
            _full_spec((1, h)),
            pl.BlockSpec(memory_space=pltpu.SMEM),
        ],
        out_specs=_row_spec(bn, 1),
        out_shape=jax.ShapeDtypeStruct((n, 1), jnp.float32),
    )(h2, stats, g.reshape(1, h), be.reshape(1, h), w3.reshape(1, h), b3)
    return out


def _segment_sum(table, src, dst, num_segments):
    n_src, d = table.shape
    assert d % 16 == 0
    (e,) = src.shape
    n = num_segments
    bsz = 96
    wr = (49152 // d) // 32 * 32
    span = 16 * wr
    chunks = max(2, -(-n // span))
    if chunks % 2:
        chunks += 1
    kpc = chunks // 2
    n_pad = chunks * span
    nw = chunks * 16

    perm = jnp.argsort(dst)
    dsts = jnp.pad(dst[perm], (0, bsz), constant_values=jnp.int32(2**30))
    srcs = jnp.pad(src[perm], (0, bsz))
    bounds = jnp.searchsorted(
        dsts, jnp.arange(nw + 1, dtype=jnp.int32) * wr).astype(jnp.int32)
    bounds = jnp.pad(bounds, (0, 31))
    nbv = -(-bounds.shape[0] // 16) * 16
    bounds = bounds[:nbv] if bounds.shape[0] >= nbv else jnp.pad(
        bounds, (0, nbv - bounds.shape[0]))

    mesh = plsc.VectorSubcoreMesh(core_axis_name="c", subcore_axis_name="s")

    @functools.partial(
        pl.kernel,
        out_type=jax.ShapeDtypeStruct((n_pad, d), jnp.float32),
        scratch_types=[
            pltpu.VMEM((nbv,), jnp.int32),
            pltpu.VMEM((bsz,), jnp.int32),
            pltpu.VMEM((bsz,), jnp.int32),
            pltpu.VMEM((bsz, d), jnp.float32),
            pltpu.VMEM((wr + 8, d), jnp.float32),
            pltpu.SemaphoreType.DMA,
        ],
        mesh=mesh,
    )
    def k(table_hbm, src_hbm, dst_hbm, bounds_hbm, out_hbm,
          bv, sbat, dbat, gbuf, acc, sem):
        cid = lax.axis_index("c")
        sid = lax.axis_index("s")
        pltpu.sync_copy(bounds_hbm, bv)

        def chunk_body(kk, _):
            w = (2 * kk + cid) * 16 + sid
            base = w * wr

            def zrow(rr, _):
                for cc in range(d // 16):
                    acc[rr, pl.ds(cc * 16, 16)] = jnp.zeros((16,),
                                                            jnp.float32)
                return 0
            lax.fori_loop(0, wr + 8, zrow, 0)

            bpair = bv[pl.ds(w, 16)]
            b_lo = bpair[0]
            b_hi = bpair[1]
            abase = (b_lo // 8) * 8
            nb = (b_hi - abase + bsz - 1) // bsz

            def bbody(j, _):
                eoff = abase + j * bsz
                pltpu.sync_copy(src_hbm.at[pl.ds(eoff, bsz)], sbat)
                pltpu.sync_copy(dst_hbm.at[pl.ds(eoff, bsz)], dbat)
                pltpu.async_copy(table_hbm.at[sbat], gbuf, sem).wait()

                def gadd(g, _):
                    dvec = dbat[pl.ds(g * 16, 16)]
                    dr = jnp.where((dvec >= base) & (dvec < base + wr),
                                   dvec - base, wr)
                    for jj in range(16):
                        dj = dr[jj]
                        for cc in range(d // 16):
                            sl = pl.ds(cc * 16, 16)
                            acc[dj, sl] += gbuf[g * 16 + jj, sl]
                    return 0
                lax.fori_loop(0, bsz // 16, gadd, 0)
                return 0
            lax.fori_loop(0, nb, bbody, 0)

            def wrow(z, _):
                pltpu.sync_copy(acc.at[pl.ds(z * 32, 32)],
                                out_hbm.at[pl.ds(base + z * 32, 32)])
                return 0
            lax.fori_loop(0, wr // 32, wrow, 0)
            return 0

        lax.fori_loop(0, kpc, chunk_body, 0)

    out = k(table, srcs, dsts, bounds)
    return out[:n] if n_pad != n else out


_NW = 32


def _gather_rows(table, idx):
    n, d = table.shape
    (p,) = idx.shape
    batch = 128
    per_w = -(-p // _NW)
    per_w = -(-per_w // batch) * batch
    p_pad = per_w * _NW
    idx_p = jnp.pad(idx, (0, p_pad - p)) if p_pad != p else idx
    nb = per_w // batch
    mesh = plsc.VectorSubcoreMesh(core_axis_name="c", subcore_axis_name="s")

    @functools.partial(
        pl.kernel,
        out_type=jax.ShapeDtypeStruct((p_pad, d), jnp.float32),
        scratch_types=[
            pltpu.VMEM((batch,), jnp.int32),
            pltpu.VMEM((batch,), jnp.int32),
            pltpu.VMEM((batch, d), jnp.float32),
            pltpu.VMEM((batch, d), jnp.float32),
            pltpu.SemaphoreType.DMA,
            pltpu.SemaphoreType.DMA,
        ],
        mesh=mesh,
    )
    def k(table_hbm, idx_hbm, out_hbm, idx0, idx1, rows0, rows1, sem0, sem1):
        wid = lax.axis_index("s") * 2 + lax.axis_index("c")
        base = wid * per_w
        idx_v = (idx0, idx1)
        rows_v = (rows0, rows1)
        sems = (sem0, sem1)
        cps = [None, None]
        for j in range(nb + 1):
            cur = j % 2
            if j < nb:
                pltpu.sync_copy(idx_hbm.at[pl.ds(base + j * batch, batch)],
                                idx_v[cur])
                cps[cur] = pltpu.async_copy(table_hbm.at[idx_v[cur]],
                                            rows_v[cur], sems[cur])
            if j > 0:
                prev = (j - 1) % 2
                cps[prev].wait()
                pltpu.sync_copy(
                    rows_v[prev],
                    out_hbm.at[pl.ds(base + (j - 1) * batch, batch)])

    out = k(table, idx_p)
    return out[:p] if p_pad != p else out


def kernel(x_op, x_machine, ei_om_src, ei_om_dst, ei_mo_src, ei_mo_dst,
           pair_machine, pair_op, params):
    n_op = x_op.shape[0]
    n_ma = x_machine.shape[0]
    n_pair = pair_op.shape[0]

    x = {'operation': x_op, 'machine': x_machine}
    resid = None
    for l in range(len(params['layers'])):
        lp = params['layers'][l]
        msg_ma = _segment_sum(x['operation'], ei_om_src, ei_om_dst, n_ma)
        msg_op = _segment_sum(x['machine'], ei_mo_src, ei_mo_dst, n_op)
        new = {}
        for t, msg in (('machine', msg_ma), ('operation', msg_op)):
            p = lp[t]
            nrows = x[t].shape[0]
            h1, st = _linear1(x[t], msg, p['W1'], p['b1'], p['eps'],
                              want_stats=True)
            out, _ = _bn_relu_linear(
                h1, st, nrows, p['g1'], p['be1'], p['W2'], p['b2'],
                res=(resid[t] if resid is not None else None))
            new[t] = out
        resid = new
        x = new

    sp = params['score']
    hh = x['machine'].shape[1]
    w1_ma = sp['W1'][:hh]
    w1_op = sp['W1'][hh:]
    y_ma, _ = _linear1(x['machine'], None, w1_ma, sp['b1'], None,
                       want_stats=False)
    y_op, _ = _linear1(x['operation'], None, w1_op, None, None,
                       want_stats=False)
    gm = _gather_rows(y_ma, pair_machine)
    go = _gather_rows(y_op, pair_op)
    h1, st1 = _add_stats(gm, go)
    h2, st2 = _bn_relu_linear(h1, st1, n_pair, sp['g1'], sp['be1'],
                              sp['W2'], sp['b2'], want_stats=True)
    out = _bn_relu_final(h2, st2, n_pair, sp['g2'], sp['be2'],
                         sp['W3'], sp['b3'])
    return out[:, 0]


# speedup vs baseline: 1.1441x; 1.1441x over previous
import functools

import jax
import jax.numpy as jnp
from jax import lax
from jax.experimental import pallas as pl
from jax.experimental.pallas import tpu as pltpu
from jax.experimental.pallas import tpu_sc as plsc

_BN_EPS = 1e-5


def _pick_bn(n, cap=2048):
    for bn in (2048, 2000, 1600, 1280, 1024, 1000, 800, 640, 512, 500,
               400, 320, 256, 250, 200, 160, 128, 100, 80, 64, 50, 40,
               32, 25, 16, 8):
        if bn <= cap and n % bn == 0 and bn % 8 == 0:
            return bn
    return n


def _row_spec(bn, d):
    return pl.BlockSpec((bn, d), lambda i: (i, 0))


def _full_spec(shape):
    return pl.BlockSpec(shape, lambda i: tuple(0 for _ in shape))


def _linear1(x, msg, w, b, eps, want_stats):
    n, din = x.shape
    h = w.shape[1]
    bn = _pick_bn(n)
    grid = n // bn

    has_msg = msg is not None
    has_b = b is not None
    has_eps = eps is not None

    def body(*refs):
        it = iter(refs)
        x_ref = next(it)
        msg_ref = next(it) if has_msg else None
        w_ref = next(it)
        b_ref = next(it) if has_b else None
        eps_ref = next(it) if has_eps else None
        o_ref = next(it)
        st_ref = next(it) if want_stats else None
        acc_ref = next(it) if want_stats else None

        g = x_ref[...]
        if has_eps:
            g = (1.0 + eps_ref[0]) * g
        if has_msg:
            g = g + msg_ref[...]
        hh = jnp.dot(g, w_ref[...], preferred_element_type=jnp.float32)
        if has_b:
            hh = hh + b_ref[...]
        o_ref[...] = hh
        if want_stats:
            i = pl.program_id(0)

            @pl.when(i == 0)
            def _():
                acc_ref[...] = jnp.zeros_like(acc_ref)

            acc_ref[0:1, :] += jnp.sum(hh, axis=0, keepdims=True)
            acc_ref[1:2, :] += jnp.sum(hh * hh, axis=0, keepdims=True)

            @pl.when(i == grid - 1)
            def _():
                st_ref[...] = acc_ref[...]

    in_specs = [_row_spec(bn, din)]
    args = [x]
    if has_msg:
        in_specs.append(_row_spec(bn, din))
        args.append(msg)
    in_specs.append(_full_spec((din, h)))
    args.append(w)
    if has_b:
        in_specs.append(_full_spec((1, h)))
        args.append(b.reshape(1, h))
    if has_eps:
        in_specs.append(pl.BlockSpec(memory_space=pltpu.SMEM))
        args.append(eps.reshape(1))

    out_shape = [jax.ShapeDtypeStruct((n, h), jnp.float32)]
    out_specs = [_row_spec(bn, h)]
    scratch = []
    if want_stats:
        out_shape.append(jax.ShapeDtypeStruct((2, h), jnp.float32))
        out_specs.append(_full_spec((2, h)))
        scratch.append(pltpu.VMEM((2, h), jnp.float32))

    res = pl.pallas_call(
        body,
        grid=(grid,),
        in_specs=in_specs,
        out_specs=out_specs if len(out_specs) > 1 else out_specs,
        out_shape=out_shape,
        scratch_shapes=scratch,
    )(*args)
    if want_stats:
        return res[0], res[1]
    return res[0], None


def _bn_relu_linear(h1, stats, nrows, g, be, w, b, res=None, want_stats=False):
    n, h = h1.shape
    h2 = w.shape[1]
    bn = _pick_bn(n)
    grid = n // bn
    has_res = res is not None
    inv_n = 1.0 / float(nrows)

    def body(*refs):
        it = iter(refs)
        h_ref = next(it)
        st_ref = next(it)
        g_ref = next(it)
        be_ref = next(it)
        w_ref = next(it)
        b_ref = next(it)
        res_ref = next(it) if has_res else None
        o_ref = next(it)
        sto_ref = next(it) if want_stats else None
        acc_ref = next(it) if want_stats else None

        mu = st_ref[0:1, :] * inv_n
        var = st_ref[1:2, :] * inv_n - mu * mu
        scale = g_ref[...] * jax.lax.rsqrt(var + _BN_EPS)
        shift = be_ref[...] - mu * scale
        a = jnp.maximum(h_ref[...] * scale + shift, 0.0)
        out = jnp.dot(a, w_ref[...], preferred_element_type=jnp.float32)
        out = out + b_ref[...]
        if has_res:
            out = out + res_ref[...]
        o_ref[...] = out
        if want_stats:
            i = pl.program_id(0)

            @pl.when(i == 0)
            def _():
                acc_ref[...] = jnp.zeros_like(acc_ref)

            acc_ref[0:1, :] += jnp.sum(out, axis=0, keepdims=True)
            acc_ref[1:2, :] += jnp.sum(out * out, axis=0, keepdims=True)

            @pl.when(i == grid - 1)
            def _():
                sto_ref[...] = acc_ref[...]

    in_specs = [
        _row_spec(bn, h),
        _full_spec((2, h)),
        _full_spec((1, h)),
        _full_spec((1, h)),
        _full_spec((h, h2)),
        _full_spec((1, h2)),
    ]
    args = [h1, stats, g.reshape(1, h), be.reshape(1, h), w, b.reshape(1, h2)]
    if has_res:
        in_specs.append(_row_spec(bn, h2))
        args.append(res)

    out_shape = [jax.ShapeDtypeStruct((n, h2), jnp.float32)]
    out_specs = [_row_spec(bn, h2)]
    scratch = []
    if want_stats:
        out_shape.append(jax.ShapeDtypeStruct((2, h2), jnp.float32))
        out_specs.append(_full_spec((2, h2)))
        scratch.append(pltpu.VMEM((2, h2), jnp.float32))

    res_out = pl.pallas_call(
        body,
        grid=(grid,),
        in_specs=in_specs,
        out_specs=out_specs,
        out_shape=out_shape,
        scratch_shapes=scratch,
    )(*args)
    if want_stats:
        return res_out[0], res_out[1]
    return res_out[0], None


def _add_stats(a, b):
    n, h = a.shape
    bn = _pick_bn(n)
    grid = n // bn

    def body(a_ref, b_ref, o_ref, st_ref, acc_ref):
        out = a_ref[...] + b_ref[...]
        o_ref[...] = out
        i = pl.program_id(0)

        @pl.when(i == 0)
        def _():
            acc_ref[...] = jnp.zeros_like(acc_ref)

        acc_ref[0:1, :] += jnp.sum(out, axis=0, keepdims=True)
        acc_ref[1:2, :] += jnp.sum(out * out, axis=0, keepdims=True)

        @pl.when(i == grid - 1)
        def _():
            st_ref[...] = acc_ref[...]

    out, st = pl.pallas_call(
        body,
        grid=(grid,),
        in_specs=[_row_spec(bn, h), _row_spec(bn, h)],
        out_specs=[_row_spec(bn, h), _full_spec((2, h))],
        out_shape=[jax.ShapeDtypeStruct((n, h), jnp.float32),
                   jax.ShapeDtypeStruct((2, h), jnp.float32)],
        scratch_shapes=[pltpu.VMEM((2, h), jnp.float32)],
    )(a, b)
    return out, st


def _bn_relu_final(h2, stats, nrows, g, be, w3, b3):
    n, h = h2.shape
    bn = _pick_bn(n)
    grid = n // bn
    inv_n = 1.0 / float(nrows)

    def body(h_ref, st_ref, g_ref, be_ref, w_ref, b_ref, o_ref):
        mu = st_ref[0:1, :] * inv_n
        var = st_ref[1:2, :] * inv_n - mu * mu
        scale = g_ref[...] * jax.lax.rsqrt(var + _BN_EPS)
        shift = be_ref[...] - mu * scale
        a = jnp.maximum(h_ref[...] * scale + shift, 0.0)
        o_ref[...] = jnp.sum(a * w_ref[...], axis=1, keepdims=True) + b_ref[0]

    out = pl.pallas_call(
        body,
        grid=(grid,),
        in_specs=[
            _row_spec(bn, h),
            _full_spec((2, h)),
            _full_spec((1, h)),
            _full_spec((1, h)),
            _full_spec((1, h)),
            pl.BlockSpec(memory_space=pltpu.SMEM),
        ],
        out_specs=_row_spec(bn, 1),
        out_shape=jax.ShapeDtypeStruct((n, 1), jnp.float32),
    )(h2, stats, g.reshape(1, h), be.reshape(1, h), w3.reshape(1, h), b3)
    return out


def _segment_sum(table, src, dst, num_segments):
    n_src, d = table.shape
    assert d % 16 == 0
    (e,) = src.shape
    n = num_segments
    bsz = 96
    wr = (49152 // d) // 32 * 32
    span = 16 * wr
    chunks = max(2, -(-n // span))
    if chunks % 2:
        chunks += 1
    kpc = chunks // 2
    n_pad = chunks * span
    nw = chunks * 16

    perm = jnp.argsort(dst)
    dsts = jnp.pad(dst[perm], (0, bsz), constant_values=jnp.int32(2**30))
    srcs = jnp.pad(src[perm], (0, bsz))
    bounds = jnp.searchsorted(
        dsts, jnp.arange(nw + 1, dtype=jnp.int32) * wr).astype(jnp.int32)
    bounds = jnp.pad(bounds, (0, 31))
    nbv = -(-bounds.shape[0] // 16) * 16
    bounds = bounds[:nbv] if bounds.shape[0] >= nbv else jnp.pad(
        bounds, (0, nbv - bounds.shape[0]))

    mesh = plsc.VectorSubcoreMesh(core_axis_name="c", subcore_axis_name="s")

    @functools.partial(
        pl.kernel,
        out_type=jax.ShapeDtypeStruct((n_pad, d), jnp.float32),
        scratch_types=[
            pltpu.VMEM((nbv,), jnp.int32),
            pltpu.VMEM((bsz,), jnp.int32),
            pltpu.VMEM((bsz,), jnp.int32),
            pltpu.VMEM((bsz,), jnp.int32),
            pltpu.VMEM((bsz,), jnp.int32),
            pltpu.VMEM((bsz, d), jnp.float32),
            pltpu.VMEM((bsz, d), jnp.float32),
            pltpu.VMEM((wr + 8, d), jnp.float32),
            pltpu.SemaphoreType.DMA,
            pltpu.SemaphoreType.DMA,
        ],
        mesh=mesh,
    )
    def k(table_hbm, src_hbm, dst_hbm, bounds_hbm, out_hbm,
          bv, sbat0, sbat1, dbat0, dbat1, gbuf0, gbuf1, acc, sem0, sem1):
        cid = lax.axis_index("c")
        sid = lax.axis_index("s")
        sbats = (sbat0, sbat1)
        dbats = (dbat0, dbat1)
        gbufs = (gbuf0, gbuf1)
        sems = (sem0, sem1)
        pltpu.sync_copy(bounds_hbm, bv)

        def chunk_body(kk, _):
            w = (2 * kk + cid) * 16 + sid
            base = w * wr

            def zrow(rr, _):
                for cc in range(d // 16):
                    acc[rr, pl.ds(cc * 16, 16)] = jnp.zeros((16,),
                                                            jnp.float32)
                return 0
            lax.fori_loop(0, wr + 8, zrow, 0)

            bpair = bv[pl.ds(w, 16)]
            b_lo = bpair[0]
            b_hi = bpair[1]
            abase = (b_lo // 8) * 8
            nb = (b_hi - abase + bsz - 1) // bsz

            def fetch(b, par):
                eoff = abase + b * bsz
                pltpu.sync_copy(src_hbm.at[pl.ds(eoff, bsz)], sbats[par])
                pltpu.sync_copy(dst_hbm.at[pl.ds(eoff, bsz)], dbats[par])
                pltpu.async_copy(table_hbm.at[sbats[par]], gbufs[par],
                                 sems[par])

            for par in (0, 1):
                @pl.when(par < nb)
                def _(par=par):
                    fetch(jnp.int32(par), par)

            def bpair(j, _):
                for par in (0, 1):
                    b = 2 * j + par

                    @pl.when(b < nb)
                    def _(b=b, par=par):
                        pltpu.make_async_copy(table_hbm.at[sbats[par]],
                                              gbufs[par], sems[par]).wait()

                        def gadd(g, _):
                            dvec = dbats[par][pl.ds(g * 16, 16)]
                            dr = jnp.where(
                                (dvec >= base) & (dvec < base + wr),
                                dvec - base, wr)
                            for jj in range(16):
                                dj = dr[jj]
                                for cc in range(d // 16):
                                    sl = pl.ds(cc * 16, 16)
                                    acc[dj, sl] += gbufs[par][g * 16 + jj,
                                                              sl]
                            return 0
                        lax.fori_loop(0, bsz // 16, gadd, 0)

                        @pl.when(b + 2 < nb)
                        def _():
                            fetch(b + 2, par)
                return 0
            lax.fori_loop(0, (nb + 1) // 2, bpair, 0)

            def wrow(z, _):
                pltpu.sync_copy(acc.at[pl.ds(z * 32, 32)],
                                out_hbm.at[pl.ds(base + z * 32, 32)])
                return 0
            lax.fori_loop(0, wr // 32, wrow, 0)
            return 0

        lax.fori_loop(0, kpc, chunk_body, 0)

    out = k(table, srcs, dsts, bounds)
    return out[:n] if n_pad != n else out


_NW = 32


def _gather_rows(table, idx):
    n, d = table.shape
    (p,) = idx.shape
    batch = 128
    per_w = -(-p // _NW)
    per_w = -(-per_w // batch) * batch
    p_pad = per_w * _NW
    idx_p = jnp.pad(idx, (0, p_pad - p)) if p_pad != p else idx
    nb = per_w // batch
    mesh = plsc.VectorSubcoreMesh(core_axis_name="c", subcore_axis_name="s")

    @functools.partial(
        pl.kernel,
        out_type=jax.ShapeDtypeStruct((p_pad, d), jnp.float32),
        scratch_types=[
            pltpu.VMEM((batch,), jnp.int32),
            pltpu.VMEM((batch,), jnp.int32),
            pltpu.VMEM((batch, d), jnp.float32),
            pltpu.VMEM((batch, d), jnp.float32),
            pltpu.SemaphoreType.DMA,
            pltpu.SemaphoreType.DMA,
        ],
        mesh=mesh,
    )
    def k(table_hbm, idx_hbm, out_hbm, idx0, idx1, rows0, rows1, sem0, sem1):
        wid = lax.axis_index("s") * 2 + lax.axis_index("c")
        base = wid * per_w
        idx_v = (idx0, idx1)
        rows_v = (rows0, rows1)
        sems = (sem0, sem1)
        cps = [None, None]
        for j in range(nb + 1):
            cur = j % 2
            if j < nb:
                pltpu.sync_copy(idx_hbm.at[pl.ds(base + j * batch, batch)],
                                idx_v[cur])
                cps[cur] = pltpu.async_copy(table_hbm.at[idx_v[cur]],
                                            rows_v[cur], sems[cur])
            if j > 0:
                prev = (j - 1) % 2
                cps[prev].wait()
                pltpu.sync_copy(
                    rows_v[prev],
                    out_hbm.at[pl.ds(base + (j - 1) * batch, batch)])

    out = k(table, idx_p)
    return out[:p] if p_pad != p else out


def kernel(x_op, x_machine, ei_om_src, ei_om_dst, ei_mo_src, ei_mo_dst,
           pair_machine, pair_op, params):
    n_op = x_op.shape[0]
    n_ma = x_machine.shape[0]
    n_pair = pair_op.shape[0]

    x = {'operation': x_op, 'machine': x_machine}
    resid = None
    for l in range(len(params['layers'])):
        lp = params['layers'][l]
        msg_ma = _segment_sum(x['operation'], ei_om_src, ei_om_dst, n_ma)
        msg_op = _segment_sum(x['machine'], ei_mo_src, ei_mo_dst, n_op)
        new = {}
        for t, msg in (('machine', msg_ma), ('operation', msg_op)):
            p = lp[t]
            nrows = x[t].shape[0]
            h1, st = _linear1(x[t], msg, p['W1'], p['b1'], p['eps'],
                              want_stats=True)
            out, _ = _bn_relu_linear(
                h1, st, nrows, p['g1'], p['be1'], p['W2'], p['b2'],
                res=(resid[t] if resid is not None else None))
            new[t] = out
        resid = new
        x = new

    sp = params['score']
    hh = x['machine'].shape[1]
    w1_ma = sp['W1'][:hh]
    w1_op = sp['W1'][hh:]
    y_ma, _ = _linear1(x['machine'], None, w1_ma, sp['b1'], None,
                       want_stats=False)
    y_op, _ = _linear1(x['operation'], None, w1_op, None, None,
                       want_stats=False)
    gm = _gather_rows(y_ma, pair_machine)
    go = _gather_rows(y_op, pair_op)
    h1, st1 = _add_stats(gm, go)
    h2, st2 = _bn_relu_linear(h1, st1, n_pair, sp['g1'], sp['be1'],
                              sp['W2'], sp['b2'], want_stats=True)
    out = _bn_relu_final(h2, st2, n_pair, sp['g2'], sp['be2'],
                         sp['W3'], sp['b3'])
    return out[:, 0]


# speedup vs baseline: 1.2904x; 1.1280x over previous
import functools

import jax
import jax.numpy as jnp
from jax import lax
from jax.experimental import pallas as pl
from jax.experimental.pallas import tpu as pltpu
from jax.experimental.pallas import tpu_sc as plsc

_BN_EPS = 1e-5


def _pick_bn(n, cap=2048):
    for bn in (2048, 2000, 1600, 1280, 1024, 1000, 800, 640, 512, 500,
               400, 320, 256, 250, 200, 160, 128, 100, 80, 64, 50, 40,
               32, 25, 16, 8):
        if bn <= cap and n % bn == 0 and bn % 8 == 0:
            return bn
    return n


def _row_spec(bn, d):
    return pl.BlockSpec((bn, d), lambda i: (i, 0))


def _full_spec(shape):
    return pl.BlockSpec(shape, lambda i: tuple(0 for _ in shape))


def _linear1(x, msg, w, b, eps, want_stats):
    n, din = x.shape
    h = w.shape[1]
    bn = _pick_bn(n)
    grid = n // bn

    has_msg = msg is not None
    has_b = b is not None
    has_eps = eps is not None

    def body(*refs):
        it = iter(refs)
        x_ref = next(it)
        msg_ref = next(it) if has_msg else None
        w_ref = next(it)
        b_ref = next(it) if has_b else None
        eps_ref = next(it) if has_eps else None
        o_ref = next(it)
        st_ref = next(it) if want_stats else None
        acc_ref = next(it) if want_stats else None

        g = x_ref[...]
        if has_eps:
            g = (1.0 + eps_ref[0]) * g
        if has_msg:
            g = g + msg_ref[...]
        hh = jnp.dot(g, w_ref[...], preferred_element_type=jnp.float32)
        if has_b:
            hh = hh + b_ref[...]
        o_ref[...] = hh
        if want_stats:
            i = pl.program_id(0)

            @pl.when(i == 0)
            def _():
                acc_ref[...] = jnp.zeros_like(acc_ref)

            acc_ref[0:1, :] += jnp.sum(hh, axis=0, keepdims=True)
            acc_ref[1:2, :] += jnp.sum(hh * hh, axis=0, keepdims=True)

            @pl.when(i == grid - 1)
            def _():
                st_ref[...] = acc_ref[...]

    in_specs = [_row_spec(bn, din)]
    args = [x]
    if has_msg:
        in_specs.append(_row_spec(bn, din))
        args.append(msg)
    in_specs.append(_full_spec((din, h)))
    args.append(w)
    if has_b:
        in_specs.append(_full_spec((1, h)))
        args.append(b.reshape(1, h))
    if has_eps:
        in_specs.append(pl.BlockSpec(memory_space=pltpu.SMEM))
        args.append(eps.reshape(1))

    out_shape = [jax.ShapeDtypeStruct((n, h), jnp.float32)]
    out_specs = [_row_spec(bn, h)]
    scratch = []
    if want_stats:
        out_shape.append(jax.ShapeDtypeStruct((2, h), jnp.float32))
        out_specs.append(_full_spec((2, h)))
        scratch.append(pltpu.VMEM((2, h), jnp.float32))

    res = pl.pallas_call(
        body,
        grid=(grid,),
        in_specs=in_specs,
        out_specs=out_specs if len(out_specs) > 1 else out_specs,
        out_shape=out_shape,
        scratch_shapes=scratch,
    )(*args)
    if want_stats:
        return res[0], res[1]
    return res[0], None


def _bn_relu_linear(h1, stats, nrows, g, be, w, b, res=None, want_stats=False):
    n, h = h1.shape
    h2 = w.shape[1]
    bn = _pick_bn(n)
    grid = n // bn
    has_res = res is not None
    inv_n = 1.0 / float(nrows)

    def body(*refs):
        it = iter(refs)
        h_ref = next(it)
        st_ref = next(it)
        g_ref = next(it)
        be_ref = next(it)
        w_ref = next(it)
        b_ref = next(it)
        res_ref = next(it) if has_res else None
        o_ref = next(it)
        sto_ref = next(it) if want_stats else None
        acc_ref = next(it) if want_stats else None

        mu = st_ref[0:1, :] * inv_n
        var = st_ref[1:2, :] * inv_n - mu * mu
        scale = g_ref[...] * jax.lax.rsqrt(var + _BN_EPS)
        shift = be_ref[...] - mu * scale
        a = jnp.maximum(h_ref[...] * scale + shift, 0.0)
        out = jnp.dot(a, w_ref[...], preferred_element_type=jnp.float32)
        out = out + b_ref[...]
        if has_res:
            out = out + res_ref[...]
        o_ref[...] = out
        if want_stats:
            i = pl.program_id(0)

            @pl.when(i == 0)
            def _():
                acc_ref[...] = jnp.zeros_like(acc_ref)

            acc_ref[0:1, :] += jnp.sum(out, axis=0, keepdims=True)
            acc_ref[1:2, :] += jnp.sum(out * out, axis=0, keepdims=True)

            @pl.when(i == grid - 1)
            def _():
                sto_ref[...] = acc_ref[...]

    in_specs = [
        _row_spec(bn, h),
        _full_spec((2, h)),
        _full_spec((1, h)),
        _full_spec((1, h)),
        _full_spec((h, h2)),
        _full_spec((1, h2)),
    ]
    args = [h1, stats, g.reshape(1, h), be.reshape(1, h), w, b.reshape(1, h2)]
    if has_res:
        in_specs.append(_row_spec(bn, h2))
        args.append(res)

    out_shape = [jax.ShapeDtypeStruct((n, h2), jnp.float32)]
    out_specs = [_row_spec(bn, h2)]
    scratch = []
    if want_stats:
        out_shape.append(jax.ShapeDtypeStruct((2, h2), jnp.float32))
        out_specs.append(_full_spec((2, h2)))
        scratch.append(pltpu.VMEM((2, h2), jnp.float32))

    res_out = pl.pallas_call(
        body,
        grid=(grid,),
        in_specs=in_specs,
        out_specs=out_specs,
        out_shape=out_shape,
        scratch_shapes=scratch,
    )(*args)
    if want_stats:
        return res_out[0], res_out[1]
    return res_out[0], None


def _add_stats(a, b):
    n, h = a.shape
    bn = _pick_bn(n)
    grid = n // bn

    def body(a_ref, b_ref, o_ref, st_ref, acc_ref):
        out = a_ref[...] + b_ref[...]
        o_ref[...] = out
        i = pl.program_id(0)

        @pl.when(i == 0)
        def _():
            acc_ref[...] = jnp.zeros_like(acc_ref)

        acc_ref[0:1, :] += jnp.sum(out, axis=0, keepdims=True)
        acc_ref[1:2, :] += jnp.sum(out * out, axis=0, keepdims=True)

        @pl.when(i == grid - 1)
        def _():
            st_ref[...] = acc_ref[...]

    out, st = pl.pallas_call(
        body,
        grid=(grid,),
        in_specs=[_row_spec(bn, h), _row_spec(bn, h)],
        out_specs=[_row_spec(bn, h), _full_spec((2, h))],
        out_shape=[jax.ShapeDtypeStruct((n, h), jnp.float32),
                   jax.ShapeDtypeStruct((2, h), jnp.float32)],
        scratch_shapes=[pltpu.VMEM((2, h), jnp.float32)],
    )(a, b)
    return out, st


def _bn_relu_final(h2, stats, nrows, g, be, w3, b3):
    n, h = h2.shape
    bn = _pick_bn(n)
    grid = n // bn
    inv_n = 1.0 / float(nrows)

    def body(h_ref, st_ref, g_ref, be_ref, w_ref, b_ref, o_ref):
        mu = st_ref[0:1, :] * inv_n
        var = st_ref[1:2, :] * inv_n - mu * mu
        scale = g_ref[...] * jax.lax.rsqrt(var + _BN_EPS)
        shift = be_ref[...] - mu * scale
        a = jnp.maximum(h_ref[...] * scale + shift, 0.0)
        o_ref[...] = jnp.sum(a * w_ref[...], axis=1, keepdims=True) + b_ref[0]

    out = pl.pallas_call(
        body,
        grid=(grid,),
        in_specs=[
            _row_spec(bn, h),
            _full_spec((2, h)),
            _full_spec((1, h)),
            _full_spec((1, h)),
            _full_spec((1, h)),
            pl.BlockSpec(memory_space=pltpu.SMEM),
        ],
        out_specs=_row_spec(bn, 1),
        out_shape=jax.ShapeDtypeStruct((n, 1), jnp.float32),
    )(h2, stats, g.reshape(1, h), be.reshape(1, h), w3.reshape(1, h), b3)
    return out


def _segment_sum(table, src, dst, num_segments):
    n_src, d = table.shape
    assert d % 16 == 0
    (e,) = src.shape
    n = num_segments
    bsz = 128
    wr = (40960 // d) // 32 * 32
    span = 16 * wr
    chunks = max(2, -(-n // span))
    if chunks % 2:
        chunks += 1
    kpc = chunks // 2
    n_pad = chunks * span
    nw = chunks * 16

    perm = jnp.argsort(dst)
    dsts = jnp.pad(dst[perm], (0, bsz), constant_values=jnp.int32(2**30))
    srcs = jnp.pad(src[perm], (0, bsz))
    bounds = jnp.searchsorted(
        dsts, jnp.arange(nw + 1, dtype=jnp.int32) * wr).astype(jnp.int32)
    bounds = jnp.pad(bounds, (0, 31))
    nbv = -(-bounds.shape[0] // 16) * 16
    bounds = bounds[:nbv] if bounds.shape[0] >= nbv else jnp.pad(
        bounds, (0, nbv - bounds.shape[0]))

    mesh = plsc.VectorSubcoreMesh(core_axis_name="c", subcore_axis_name="s")

    @functools.partial(
        pl.kernel,
        out_type=jax.ShapeDtypeStruct((n_pad, d), jnp.float32),
        scratch_types=[
            pltpu.VMEM((nbv,), jnp.int32),
            pltpu.VMEM((bsz,), jnp.int32),
            pltpu.VMEM((bsz,), jnp.int32),
            pltpu.VMEM((bsz,), jnp.int32),
            pltpu.VMEM((bsz,), jnp.int32),
            pltpu.VMEM((bsz, d), jnp.float32),
            pltpu.VMEM((bsz, d), jnp.float32),
            pltpu.VMEM((wr + 8, d), jnp.float32),
            pltpu.SemaphoreType.DMA,
            pltpu.SemaphoreType.DMA,
        ],
        mesh=mesh,
    )
    def k(table_hbm, src_hbm, dst_hbm, bounds_hbm, out_hbm,
          bv, sbat0, sbat1, dbat0, dbat1, gbuf0, gbuf1, acc, sem0, sem1):
        cid = lax.axis_index("c")
        sid = lax.axis_index("s")
        sbats = (sbat0, sbat1)
        dbats = (dbat0, dbat1)
        gbufs = (gbuf0, gbuf1)
        sems = (sem0, sem1)
        pltpu.sync_copy(bounds_hbm, bv)

        def chunk_body(kk, _):
            w = (2 * kk + cid) * 16 + sid
            base = w * wr

            def zrow(rr, _):
                for cc in range(d // 16):
                    acc[rr, pl.ds(cc * 16, 16)] = jnp.zeros((16,),
                                                            jnp.float32)
                return 0
            lax.fori_loop(0, wr + 8, zrow, 0)

            bpair = bv[pl.ds(w, 16)]
            b_lo = bpair[0]
            b_hi = bpair[1]
            abase = (b_lo // 8) * 8
            nb = (b_hi - abase + bsz - 1) // bsz

            def fetch(b, par):
                eoff = abase + b * bsz
                pltpu.sync_copy(src_hbm.at[pl.ds(eoff, bsz)], sbats[par])
                pltpu.sync_copy(dst_hbm.at[pl.ds(eoff, bsz)], dbats[par])
                pltpu.async_copy(table_hbm.at[sbats[par]], gbufs[par],
                                 sems[par])

            for par in (0, 1):
                @pl.when(par < nb)
                def _(par=par):
                    fetch(jnp.int32(par), par)

            def bpair(j, _):
                for par in (0, 1):
                    b = 2 * j + par

                    @pl.when(b < nb)
                    def _(b=b, par=par):
                        pltpu.make_async_copy(table_hbm.at[sbats[par]],
                                              gbufs[par], sems[par]).wait()

                        def gadd(g, _):
                            dvec = dbats[par][pl.ds(g * 16, 16)]
                            dr = jnp.where(
                                (dvec >= base) & (dvec < base + wr),
                                dvec - base, wr)
                            for jj in range(16):
                                dj = dr[jj]
                                for cc in range(d // 16):
                                    sl = pl.ds(cc * 16, 16)
                                    acc[dj, sl] += gbufs[par][g * 16 + jj,
                                                              sl]
                            return 0
                        lax.fori_loop(0, bsz // 16, gadd, 0)

                        @pl.when(b + 2 < nb)
                        def _():
                            fetch(b + 2, par)
                return 0
            lax.fori_loop(0, (nb + 1) // 2, bpair, 0)

            def wrow(z, _):
                pltpu.sync_copy(acc.at[pl.ds(z * 32, 32)],
                                out_hbm.at[pl.ds(base + z * 32, 32)])
                return 0
            lax.fori_loop(0, wr // 32, wrow, 0)
            return 0

        lax.fori_loop(0, kpc, chunk_body, 0)

    out = k(table, srcs, dsts, bounds)
    return out[:n] if n_pad != n else out


_NW = 32


def _gather_rows(table, idx):
    n, d = table.shape
    (p,) = idx.shape
    batch = 128
    per_w = -(-p // _NW)
    per_w = -(-per_w // batch) * batch
    p_pad = per_w * _NW
    idx_p = jnp.pad(idx, (0, p_pad - p)) if p_pad != p else idx
    nb = per_w // batch
    mesh = plsc.VectorSubcoreMesh(core_axis_name="c", subcore_axis_name="s")

    @functools.partial(
        pl.kernel,
        out_type=jax.ShapeDtypeStruct((p_pad, d), jnp.float32),
        scratch_types=[
            pltpu.VMEM((batch,), jnp.int32),
            pltpu.VMEM((batch,), jnp.int32),
            pltpu.VMEM((batch, d), jnp.float32),
            pltpu.VMEM((batch, d), jnp.float32),
            pltpu.SemaphoreType.DMA,
            pltpu.SemaphoreType.DMA,
        ],
        mesh=mesh,
    )
    def k(table_hbm, idx_hbm, out_hbm, idx0, idx1, rows0, rows1, sem0, sem1):
        wid = lax.axis_index("s") * 2 + lax.axis_index("c")
        base = wid * per_w
        idx_v = (idx0, idx1)
        rows_v = (rows0, rows1)
        sems = (sem0, sem1)
        cps = [None, None]
        for j in range(nb + 1):
            cur = j % 2
            if j < nb:
                pltpu.sync_copy(idx_hbm.at[pl.ds(base + j * batch, batch)],
                                idx_v[cur])
                cps[cur] = pltpu.async_copy(table_hbm.at[idx_v[cur]],
                                            rows_v[cur], sems[cur])
            if j > 0:
                prev = (j - 1) % 2
                cps[prev].wait()
                pltpu.sync_copy(
                    rows_v[prev],
                    out_hbm.at[pl.ds(base + (j - 1) * batch, batch)])

    out = k(table, idx_p)
    return out[:p] if p_pad != p else out


def kernel(x_op, x_machine, ei_om_src, ei_om_dst, ei_mo_src, ei_mo_dst,
           pair_machine, pair_op, params):
    n_op = x_op.shape[0]
    n_ma = x_machine.shape[0]
    n_pair = pair_op.shape[0]

    x = {'operation': x_op, 'machine': x_machine}
    resid = None
    for l in range(len(params['layers'])):
        lp = params['layers'][l]
        msg_ma = _segment_sum(x['operation'], ei_om_src, ei_om_dst, n_ma)
        msg_op = _segment_sum(x['machine'], ei_mo_src, ei_mo_dst, n_op)
        new = {}
        for t, msg in (('machine', msg_ma), ('operation', msg_op)):
            p = lp[t]
            nrows = x[t].shape[0]
            h1, st = _linear1(x[t], msg, p['W1'], p['b1'], p['eps'],
                              want_stats=True)
            out, _ = _bn_relu_linear(
                h1, st, nrows, p['g1'], p['be1'], p['W2'], p['b2'],
                res=(resid[t] if resid is not None else None))
            new[t] = out
        resid = new
        x = new

    sp = params['score']
    hh = x['machine'].shape[1]
    w1_ma = sp['W1'][:hh]
    w1_op = sp['W1'][hh:]
    y_ma, _ = _linear1(x['machine'], None, w1_ma, sp['b1'], None,
                       want_stats=False)
    y_op, _ = _linear1(x['operation'], None, w1_op, None, None,
                       want_stats=False)
    gm = _gather_rows(y_ma, pair_machine)
    go = _gather_rows(y_op, pair_op)
    h1, st1 = _add_stats(gm, go)
    h2, st2 = _bn_relu_linear(h1, st1, n_pair, sp['g1'], sp['be1'],
                              sp['W2'], sp['b2'], want_stats=True)
    out = _bn_relu_final(h2, st2, n_pair, sp['g2'], sp['be2'],
                         sp['W3'], sp['b3'])
    return out[:, 0]
